# baseline (speedup 1.0000x reference)
import functools

import jax
import jax.numpy as jnp
from jax import lax
from jax.experimental import pallas as pl
from jax.experimental.pallas import tpu as pltpu
from jax.experimental.pallas import tpu_sc as plsc

MAX_DEG = 10
N_PER = 10000
D = 128
N = 11 * N_PER
NUM_ADJ_ROWS = MAX_DEG * (MAX_DEG + 1) // 2

NUM_WORKERS = 32
SPAN = 328
STRIDE = 312


NCHUNK = SPAN // 16
TAIL = SPAN - NCHUNK * 16


def _sc_gather_sums(atom_features, *adjs_flat):
    mesh = plsc.VectorSubcoreMesh(core_axis_name="c", subcore_axis_name="s")

    @functools.partial(
        pl.kernel,
        out_type=jax.ShapeDtypeStruct((MAX_DEG * N_PER, D), jnp.float32),
        mesh=mesh,
        compiler_params=pltpu.CompilerParams(needs_layout_passes=False),
        scratch_types=[
            pltpu.VMEM((NUM_ADJ_ROWS * SPAN,), jnp.int32),
            pltpu.VMEM((NUM_ADJ_ROWS * SPAN,), jnp.int32),
            pltpu.VMEM((SPAN, D), jnp.float32),
            pltpu.VMEM((SPAN, D), jnp.float32),
            pltpu.SemaphoreType.DMA,
            pltpu.SemaphoreType.DMA,
            pltpu.SemaphoreType.DMA,
            pltpu.SemaphoreType.DMA,
            pltpu.SemaphoreType.DMA,
            pltpu.SemaphoreType.DMA,
            pltpu.SemaphoreType.DMA,
        ],
    )
    def body(feats_hbm, a1, a2, a3, a4, a5, a6, a7, a8, a9, a10, out_hbm,
             idx_nat, idx_v, acc0, acc1, ss, sg0, sg1, sa0, sa1, sw0, sw1):
        wid = lax.axis_index("s") * 2 + lax.axis_index("c")
        base = wid * STRIDE
        adj_refs = [a1, a2, a3, a4, a5, a6, a7, a8, a9, a10]
        accs, sgs, sas, sws = [acc0, acc1], [sg0, sg1], [sa0, sa1], [sw0, sw1]
        iota = lax.iota(jnp.int32, 16)

        def gather0(d, p):
            off = d * (d - 1) // 2
            return pltpu.async_copy(
                feats_hbm.at[idx_v.at[pl.ds(off * SPAN, SPAN)]],
                accs[p], sgs[p])

        def transpose_deg(d):
            off = d * (d - 1) // 2
            nat_off = off * SPAN
            iota_d = iota * d

            def chunk(v, carry):
                vbase = nat_off + v * (16 * d)
                for j in range(d):
                    g = plsc.load_gather(idx_nat, [iota_d + (vbase + j)])
                    idx_v[pl.ds((off + j) * SPAN + v * 16, 16)] = g
                return carry

            lax.fori_loop(0, NCHUNK, chunk, 0)
            mask = iota < TAIL
            vbase = nat_off + NCHUNK * (16 * d)
            limit = NUM_ADJ_ROWS * SPAN - 1
            for j in range(d):
                src = jnp.minimum(iota_d + (vbase + j), limit)
                g = plsc.load_gather(idx_nat, [src], mask=mask)
                pos = iota + ((off + j) * SPAN + NCHUNK * 16)
                plsc.store_scatter(idx_v, [pos], g, mask=mask)

        stages = [
            pltpu.async_copy(
                adj_refs[d - 1].at[pl.ds(base * d, SPAN * d)],
                idx_nat.at[pl.ds((d * (d - 1) // 2) * SPAN, SPAN * d)], ss)
            for d in range(1, MAX_DEG + 1)
        ]
        for s in stages:
            s.wait()
        transpose_deg(1)
        g0 = {1: gather0(1, 0)}
        for d in range(2, MAX_DEG + 1):
            transpose_deg(d)
        writes = {}
        for d in range(1, MAX_DEG + 1):
            p = (d - 1) % 2
            q = d % 2
            off = d * (d - 1) // 2
            g0[d].wait()
            if d < MAX_DEG:
                if d >= 2:
                    writes[d - 1].wait()
                g0[d + 1] = gather0(d + 1, q)
            adds = [
                pltpu.async_copy(
                    feats_hbm.at[idx_v.at[pl.ds((off + j) * SPAN, SPAN)]],
                    accs[p], sas[p], add=True)
                for j in range(1, d)
            ]
            for a in adds:
                a.wait()
            writes[d] = pltpu.async_copy(
                accs[p], out_hbm.at[pl.ds((d - 1) * N_PER + base, SPAN)],
                sws[p])
        writes[MAX_DEG - 1].wait()
        writes[MAX_DEG].wait()

    return body(atom_features, *adjs_flat)


def _tc_combine(atom_features, rel_sums, W, b):
    b3 = b.reshape(b.shape[0], 1, D)
    BR = 1000
    RB = N_PER // BR

    def self_w_idx(d):
        return jnp.where(d == 0, 2 * MAX_DEG, 2 * d - 1)

    def rel_w_idx(d):
        return jnp.where(d == 0, 0, 2 * d - 2)

    def body(feat_ref, sums_ref, wself_ref, wrel_ref, bself_ref, brel_ref,
             out_ref):
        d = pl.program_id(0)
        acc = (
            jnp.dot(feat_ref[...], wself_ref[0],
                    preferred_element_type=jnp.float32)
            + bself_ref[0]
        )

        @pl.when(d > 0)
        def _():
            inv = 1.0 / d.astype(jnp.float32)
            out_ref[...] = (
                acc
                + jnp.dot(sums_ref[...], wrel_ref[0],
                          preferred_element_type=jnp.float32) * inv
                + brel_ref[0]
            )

        @pl.when(d == 0)
        def _():
            out_ref[...] = acc

    return pl.pallas_call(
        body,
        grid=(MAX_DEG + 1, RB),
        in_specs=[
            pl.BlockSpec((BR, D), lambda d, r: (d * RB + r, 0)),
            pl.BlockSpec((BR, D),
                         lambda d, r: (jnp.maximum(d - 1, 0) * RB + r, 0)),
            pl.BlockSpec((1, D, D), lambda d, r: (self_w_idx(d), 0, 0)),
            pl.BlockSpec((1, D, D), lambda d, r: (rel_w_idx(d), 0, 0)),
            pl.BlockSpec((1, 1, D), lambda d, r: (self_w_idx(d), 0, 0)),
            pl.BlockSpec((1, 1, D), lambda d, r: (rel_w_idx(d), 0, 0)),
        ],
        out_specs=pl.BlockSpec((BR, D), lambda d, r: (d * RB + r, 0)),
        out_shape=jax.ShapeDtypeStruct((N, D), jnp.float32),
    )(atom_features, rel_sums, W, W, b3, b3)


def kernel(atom_features, deg_slice, membership, deg_adj_1, deg_adj_2,
           deg_adj_3, deg_adj_4, deg_adj_5, deg_adj_6, deg_adj_7, deg_adj_8,
           deg_adj_9, deg_adj_10, W, b):
    adjs = [deg_adj_1, deg_adj_2, deg_adj_3, deg_adj_4, deg_adj_5, deg_adj_6,
            deg_adj_7, deg_adj_8, deg_adj_9, deg_adj_10]
    rel_sums = _sc_gather_sums(atom_features, *[a.reshape(-1) for a in adjs])
    return _tc_combine(atom_features, rel_sums, W, b)
